# baseline (speedup 1.0000x reference)
import jax
import jax.numpy as jnp
from jax.experimental import pallas as pl

_T = 22
_OBS = 64
_ACT = 16
_GH = 32
_EMB = 64
_BB = 512

def _split(a):
    hi = a.astype(jnp.bfloat16)
    lo = (a - hi.astype(jnp.float32)).astype(jnp.bfloat16)
    return hi, lo


def _rawdot(a, b):
    return jax.lax.dot_general(a, b, (((1,), (0,)), ((), ())),
                               preferred_element_type=jnp.float32)


def _dot_ps(a, b_hi, b_lo):
    a_hi, a_lo = _split(a)
    return _rawdot(a_hi, b_hi) + (_rawdot(a_lo, b_hi) + _rawdot(a_hi, b_lo))


def _dot(a, b):
    b_hi, b_lo = _split(b)
    return _dot_ps(a, b_hi, b_lo)


def _gru_step(h, gi, Whi, Wlo, bhh, G):
    gh = _dot_ps(h, Whi, Wlo) + bhh
    r = jax.nn.sigmoid(gi[:, :G] + gh[:, :G])
    z = jax.nn.sigmoid(gi[:, G:2 * G] + gh[:, G:2 * G])
    n = jnp.tanh(gi[:, 2 * G:] + r * gh[:, 2 * G:])
    return n + z * (h - n)


def _block_kernel(obs_ref, act_ref,
                  dtw_WihT, dtw_WhhT, dtw_bih, dtw_bhh,
                  ln_g, ln_b, m1_WT, m1_b, m2_WT, m2_b, m3_WT, m3_b,
                  f_WihT, f_bih, bk_WihT, bk_bih, fb_WhhT, fb_bhh,
                  e1_WT, e1_b, e2_WT, e2_b, mu_WT, mu_b, lv_WT, lv_b,
                  mu_ref, sigma_ref):
    BB = _BB

    def o_t(t):
        return obs_ref[:, _OBS * t:_OBS * (t + 1)]

    def a_t(t):
        return act_ref[:, _ACT * t:_ACT * (t + 1)]

    obs14 = o_t(14)
    obs13 = o_t(13)
    obs12 = o_t(12)
    obs11 = o_t(11)

    mx = jnp.max(obs14, axis=1, keepdims=True)
    ex = jnp.exp(obs14 - mx)
    p = ex / jnp.sum(ex, axis=1, keepdims=True)
    entropy = -jnp.sum(p * jnp.log(p + 1e-8), axis=1, keepdims=True)

    d0 = jnp.sqrt(jnp.sum((obs14 - obs13) ** 2, axis=1, keepdims=True))
    d1 = jnp.sqrt(jnp.sum((obs13 - obs12) ** 2, axis=1, keepdims=True))
    d2 = jnp.sqrt(jnp.sum((obs12 - obs11) ** 2, axis=1, keepdims=True))
    rate = (d0 + d1 + d2) / 3.0

    act13 = a_t(13)
    act_pad = jnp.concatenate(
        [act13, jnp.zeros((BB, _OBS - _ACT), jnp.float32)], axis=1)
    oc = obs14 - jnp.mean(obs14, axis=1, keepdims=True)
    ac = act_pad - jnp.mean(act_pad, axis=1, keepdims=True)
    denom = (jnp.sqrt(jnp.sum(oc * oc, axis=1, keepdims=True)) *
             jnp.sqrt(jnp.sum(ac * ac, axis=1, keepdims=True)) + 1e-8)
    corr = jnp.sum(oc * ac, axis=1, keepdims=True) / denom

    xs = [jnp.concatenate([o_t(t), a_t(t)], axis=1) for t in range(15)]
    X = jnp.concatenate(xs, axis=0)
    GI = _dot(X, dtw_WihT[...]) + dtw_bih[...]
    Whi, Wlo = _split(dtw_WhhT[...])
    bhh = dtw_bhh[...]
    h = jnp.zeros((BB, _GH), jnp.float32)
    for t in range(15):
        h = _gru_step(h, GI[t * BB:(t + 1) * BB, :], Whi, Wlo, bhh, _GH)

    feats = jnp.concatenate([entropy, rate, corr, h], axis=1)
    mu_f = jnp.mean(feats, axis=1, keepdims=True)
    var_f = jnp.mean((feats - mu_f) ** 2, axis=1, keepdims=True)
    fn = (feats - mu_f) / jnp.sqrt(var_f + 1e-5) * ln_g[...] + ln_b[...]
    h1 = jnp.maximum(_dot(fn, m1_WT[...]) + m1_b[...], 0.0)
    h2 = jnp.maximum(_dot(h1, m2_WT[...]) + m2_b[...], 0.0)
    logits = _dot(h2, m3_WT[...]) + m3_b[...]
    lmx = jnp.max(logits, axis=1, keepdims=True)
    iota = jax.lax.broadcasted_iota(jnp.int32, (BB, 14), 1)
    widx = jnp.min(jnp.where(logits == lmx, iota, 14), axis=1, keepdims=True)
    s_off = (widx + 1) // 2
    e_off = (widx + 2) // 2

    ch = [jnp.concatenate([o_t(7 + p), a_t(7 + p)], axis=1) for p in range(15)]
    Xf = jnp.concatenate(ch[0:8], axis=0)
    GIf = _dot(Xf, f_WihT[...]) + f_bih[...]
    Xb = jnp.concatenate([ch[14 - i] for i in range(8)], axis=0)
    GIb = _dot(Xb, bk_WihT[...]) + bk_bih[...]
    GIfb = jnp.concatenate(
        [GIf[:, 0:64], GIb[:, 0:64], GIf[:, 64:128], GIb[:, 64:128],
         GIf[:, 128:192], GIb[:, 128:192]], axis=1)
    Whi, Wlo = _split(fb_WhhT[...])
    bfb = fb_bhh[...]
    se = jnp.concatenate([jnp.broadcast_to(s_off, (BB, 64)),
                          jnp.broadcast_to(e_off, (BB, 64))], axis=1)
    hfb = jnp.zeros((BB, 2 * _EMB), jnp.float32)
    for i in range(8):
        gi = GIfb[i * BB:(i + 1) * BB, :]
        gh = _dot_ps(hfb, Whi, Wlo) + bfb
        r = jax.nn.sigmoid(gi[:, :128] + gh[:, :128])
        z = jax.nn.sigmoid(gi[:, 128:256] + gh[:, 128:256])
        n = jnp.tanh(gi[:, 256:] + r * gh[:, 256:])
        hnew = n + z * (hfb - n)
        hfb = jnp.where(se >= (7 - i), hnew, hfb)

    combined = hfb
    z1 = jnp.maximum(_dot(combined, e1_WT[...]) + e1_b[...], 0.0)
    z_enc = _dot(z1, e2_WT[...]) + e2_b[...]
    mu = _dot(z_enc, mu_WT[...]) + mu_b[...]
    lv = _dot(z_enc, lv_WT[...]) + lv_b[...]
    mu_ref[...] = mu
    sigma_ref[...] = jnp.exp(0.5 * lv)


def kernel(obs_chunk, act_chunk, dtw_Wih, dtw_Whh, dtw_bih, dtw_bhh,
           ln_g, ln_b, m1_W, m1_b, m2_W, m2_b, m3_W, m3_b,
           f_Wih, f_Whh, f_bih, f_bhh, bk_Wih, bk_Whh, bk_bih, bk_bhh,
           e1_W, e1_b, e2_W, e2_b, mu_W, mu_b, lv_W, lv_b, test_mode):
    B = obs_chunk.shape[0]
    obs2 = obs_chunk.reshape(B, _T * _OBS)
    act2 = act_chunk.reshape(B, _T * _ACT)

    fT = f_Whh.T
    bT = bk_Whh.T
    E = _EMB
    Wfb = jnp.zeros((2 * E, 6 * E), jnp.float32)
    for g in range(3):
        Wfb = Wfb.at[:E, 2 * g * E:(2 * g + 1) * E].set(fT[:, g * E:(g + 1) * E])
        Wfb = Wfb.at[E:, (2 * g + 1) * E:(2 * g + 2) * E].set(bT[:, g * E:(g + 1) * E])
    bfb = jnp.concatenate(
        [f_bhh[0:E], bk_bhh[0:E], f_bhh[E:2 * E], bk_bhh[E:2 * E],
         f_bhh[2 * E:], bk_bhh[2 * E:]]).reshape(1, 6 * E)

    weights = [
        dtw_Wih.T, dtw_Whh.T, dtw_bih.reshape(1, -1), dtw_bhh.reshape(1, -1),
        ln_g.reshape(1, -1), ln_b.reshape(1, -1),
        m1_W.T, m1_b.reshape(1, -1), m2_W.T, m2_b.reshape(1, -1),
        m3_W.T, m3_b.reshape(1, -1),
        f_Wih.T, f_bih.reshape(1, -1), bk_Wih.T, bk_bih.reshape(1, -1),
        Wfb, bfb,
        e1_W.T, e1_b.reshape(1, -1), e2_W.T, e2_b.reshape(1, -1),
        mu_W.T, mu_b.reshape(1, -1), lv_W.T, lv_b.reshape(1, -1),
    ]

    grid = (B // _BB,)
    row_in = [
        pl.BlockSpec((_BB, _T * _OBS), lambda i: (i, 0)),
        pl.BlockSpec((_BB, _T * _ACT), lambda i: (i, 0)),
    ]
    w_specs = [pl.BlockSpec(w.shape, lambda i: (0, 0)) for w in weights]
    out_specs = [
        pl.BlockSpec((_BB, _EMB), lambda i: (i, 0)),
        pl.BlockSpec((_BB, _EMB), lambda i: (i, 0)),
    ]
    mu, sigma = pl.pallas_call(
        _block_kernel,
        grid=grid,
        in_specs=row_in + w_specs,
        out_specs=out_specs,
        out_shape=[
            jax.ShapeDtypeStruct((B, _EMB), jnp.float32),
            jax.ShapeDtypeStruct((B, _EMB), jnp.float32),
        ],
    )(obs2, act2, *weights)
    return (mu, mu, sigma)


# speedup vs baseline: 1.4232x; 1.4232x over previous
import jax
import jax.numpy as jnp
from jax.experimental import pallas as pl

_T = 22
_OBS = 64
_ACT = 16
_GH = 32
_EMB = 64
_BB = 512

def _dot(a, b):
    return jax.lax.dot_general(a, b, (((1,), (0,)), ((), ())),
                               preferred_element_type=jnp.float32)


def _gru_step(h, gi, W, bhh, G):
    gh = _dot(h, W) + bhh
    r = jax.nn.sigmoid(gi[:, :G] + gh[:, :G])
    z = jax.nn.sigmoid(gi[:, G:2 * G] + gh[:, G:2 * G])
    n = jnp.tanh(gi[:, 2 * G:] + r * gh[:, 2 * G:])
    return (1.0 - z) * n + z * h


def _block_kernel(obs_ref, act_ref,
                  dtw_WihT, dtw_WhhT, dtw_bih, dtw_bhh,
                  ln_g, ln_b, m1_WT, m1_b, m2_WT, m2_b, m3_WT, m3_b,
                  f_WihT, f_WhhT, f_bih, f_bhh,
                  bk_WihT, bk_WhhT, bk_bih, bk_bhh,
                  e1_WT, e1_b, e2_WT, e2_b, mu_WT, mu_b, lv_WT, lv_b,
                  mu_ref, sigma_ref):
    BB = _BB

    def o_t(t):
        return obs_ref[:, _OBS * t:_OBS * (t + 1)]

    def a_t(t):
        return act_ref[:, _ACT * t:_ACT * (t + 1)]

    obs14 = o_t(14)
    obs13 = o_t(13)
    obs12 = o_t(12)
    obs11 = o_t(11)

    mx = jnp.max(obs14, axis=1, keepdims=True)
    ex = jnp.exp(obs14 - mx)
    p = ex / jnp.sum(ex, axis=1, keepdims=True)
    entropy = -jnp.sum(p * jnp.log(p + 1e-8), axis=1, keepdims=True)

    d0 = jnp.sqrt(jnp.sum((obs14 - obs13) ** 2, axis=1, keepdims=True))
    d1 = jnp.sqrt(jnp.sum((obs13 - obs12) ** 2, axis=1, keepdims=True))
    d2 = jnp.sqrt(jnp.sum((obs12 - obs11) ** 2, axis=1, keepdims=True))
    rate = (d0 + d1 + d2) / 3.0

    act13 = a_t(13)
    act_pad = jnp.concatenate(
        [act13, jnp.zeros((BB, _OBS - _ACT), jnp.float32)], axis=1)
    oc = obs14 - jnp.mean(obs14, axis=1, keepdims=True)
    ac = act_pad - jnp.mean(act_pad, axis=1, keepdims=True)
    denom = (jnp.sqrt(jnp.sum(oc * oc, axis=1, keepdims=True)) *
             jnp.sqrt(jnp.sum(ac * ac, axis=1, keepdims=True)) + 1e-8)
    corr = jnp.sum(oc * ac, axis=1, keepdims=True) / denom

    xs = [jnp.concatenate([o_t(t), a_t(t)], axis=1) for t in range(15)]
    X = jnp.concatenate(xs, axis=0)
    GI = _dot(X, dtw_WihT[...]) + dtw_bih[...]
    W = dtw_WhhT[...]
    bhh = dtw_bhh[...]
    h = jnp.zeros((BB, _GH), jnp.float32)
    for t in range(15):
        h = _gru_step(h, GI[t * BB:(t + 1) * BB, :], W, bhh, _GH)

    feats = jnp.concatenate([entropy, rate, corr, h], axis=1)
    mu_f = jnp.mean(feats, axis=1, keepdims=True)
    var_f = jnp.mean((feats - mu_f) ** 2, axis=1, keepdims=True)
    fn = (feats - mu_f) / jnp.sqrt(var_f + 1e-5) * ln_g[...] + ln_b[...]
    h1 = jnp.maximum(_dot(fn, m1_WT[...]) + m1_b[...], 0.0)
    h2 = jnp.maximum(_dot(h1, m2_WT[...]) + m2_b[...], 0.0)
    logits = _dot(h2, m3_WT[...]) + m3_b[...]
    lmx = jnp.max(logits, axis=1, keepdims=True)
    iota = jax.lax.broadcasted_iota(jnp.int32, (BB, 14), 1)
    widx = jnp.min(jnp.where(logits == lmx, iota, 14), axis=1, keepdims=True)
    s_off = (widx + 1) // 2
    e_off = (widx + 2) // 2

    ch = [jnp.concatenate([o_t(7 + p), a_t(7 + p)], axis=1) for p in range(15)]
    Xf = jnp.concatenate(ch[0:8], axis=0)
    GIf = _dot(Xf, f_WihT[...]) + f_bih[...]
    Xb = jnp.concatenate([ch[14 - i] for i in range(8)], axis=0)
    GIb = _dot(Xb, bk_WihT[...]) + bk_bih[...]
    fW = f_WhhT[...]
    fb = f_bhh[...]
    bW = bk_WhhT[...]
    bb = bk_bhh[...]
    hf = jnp.zeros((BB, _EMB), jnp.float32)
    hb = jnp.zeros((BB, _EMB), jnp.float32)
    for i in range(8):
        hf_new = _gru_step(hf, GIf[i * BB:(i + 1) * BB, :], fW, fb, _EMB)
        hf = jnp.where(s_off >= (7 - i), hf_new, hf)
        hb_new = _gru_step(hb, GIb[i * BB:(i + 1) * BB, :], bW, bb, _EMB)
        hb = jnp.where(e_off >= (7 - i), hb_new, hb)

    combined = jnp.concatenate([hf, hb], axis=1)
    z1 = jnp.maximum(_dot(combined, e1_WT[...]) + e1_b[...], 0.0)
    z_enc = _dot(z1, e2_WT[...]) + e2_b[...]
    mu = _dot(z_enc, mu_WT[...]) + mu_b[...]
    lv = _dot(z_enc, lv_WT[...]) + lv_b[...]
    mu_ref[...] = mu
    sigma_ref[...] = jnp.exp(0.5 * lv)


def kernel(obs_chunk, act_chunk, dtw_Wih, dtw_Whh, dtw_bih, dtw_bhh,
           ln_g, ln_b, m1_W, m1_b, m2_W, m2_b, m3_W, m3_b,
           f_Wih, f_Whh, f_bih, f_bhh, bk_Wih, bk_Whh, bk_bih, bk_bhh,
           e1_W, e1_b, e2_W, e2_b, mu_W, mu_b, lv_W, lv_b, test_mode):
    B = obs_chunk.shape[0]
    obs2 = obs_chunk.reshape(B, _T * _OBS)
    act2 = act_chunk.reshape(B, _T * _ACT)

    weights = [
        dtw_Wih.T, dtw_Whh.T, dtw_bih.reshape(1, -1), dtw_bhh.reshape(1, -1),
        ln_g.reshape(1, -1), ln_b.reshape(1, -1),
        m1_W.T, m1_b.reshape(1, -1), m2_W.T, m2_b.reshape(1, -1),
        m3_W.T, m3_b.reshape(1, -1),
        f_Wih.T, f_Whh.T, f_bih.reshape(1, -1), f_bhh.reshape(1, -1),
        bk_Wih.T, bk_Whh.T, bk_bih.reshape(1, -1), bk_bhh.reshape(1, -1),
        e1_W.T, e1_b.reshape(1, -1), e2_W.T, e2_b.reshape(1, -1),
        mu_W.T, mu_b.reshape(1, -1), lv_W.T, lv_b.reshape(1, -1),
    ]

    grid = (B // _BB,)
    row_in = [
        pl.BlockSpec((_BB, _T * _OBS), lambda i: (i, 0)),
        pl.BlockSpec((_BB, _T * _ACT), lambda i: (i, 0)),
    ]
    w_specs = [pl.BlockSpec(w.shape, lambda i: (0, 0)) for w in weights]
    out_specs = [
        pl.BlockSpec((_BB, _EMB), lambda i: (i, 0)),
        pl.BlockSpec((_BB, _EMB), lambda i: (i, 0)),
    ]
    mu, sigma = pl.pallas_call(
        _block_kernel,
        grid=grid,
        in_specs=row_in + w_specs,
        out_specs=out_specs,
        out_shape=[
            jax.ShapeDtypeStruct((B, _EMB), jnp.float32),
            jax.ShapeDtypeStruct((B, _EMB), jnp.float32),
        ],
    )(obs2, act2, *weights)
    return (mu, mu, sigma)


# speedup vs baseline: 1.6364x; 1.1499x over previous
import jax
import jax.numpy as jnp
from jax.experimental import pallas as pl

_T = 22
_OBS = 64
_ACT = 16
_GH = 32
_EMB = 64
_BB = 1024

def _dot(a, b):
    return jax.lax.dot_general(a, b, (((1,), (0,)), ((), ())),
                               preferred_element_type=jnp.float32)


def _gru_step(h, gi, W, bhh, G):
    gh = _dot(h, W) + bhh
    r = jax.nn.sigmoid(gi[:, :G] + gh[:, :G])
    z = jax.nn.sigmoid(gi[:, G:2 * G] + gh[:, G:2 * G])
    n = jnp.tanh(gi[:, 2 * G:] + r * gh[:, 2 * G:])
    return (1.0 - z) * n + z * h


def _block_kernel(obs_ref, act_ref,
                  dtw_WihT, dtw_WhhT, dtw_bih, dtw_bhh,
                  ln_g, ln_b, m1_WT, m1_b, m2_WT, m2_b, m3_WT, m3_b,
                  f_WihT, f_WhhT, f_bih, f_bhh,
                  bk_WihT, bk_WhhT, bk_bih, bk_bhh,
                  e1_WT, e1_b, e2_WT, e2_b, mu_WT, mu_b, lv_WT, lv_b,
                  mu_ref, sigma_ref):
    BB = _BB

    def o_t(t):
        return obs_ref[:, _OBS * t:_OBS * (t + 1)]

    def a_t(t):
        return act_ref[:, _ACT * t:_ACT * (t + 1)]

    obs14 = o_t(14)
    obs13 = o_t(13)
    obs12 = o_t(12)
    obs11 = o_t(11)

    mx = jnp.max(obs14, axis=1, keepdims=True)
    ex = jnp.exp(obs14 - mx)
    p = ex / jnp.sum(ex, axis=1, keepdims=True)
    entropy = -jnp.sum(p * jnp.log(p + 1e-8), axis=1, keepdims=True)

    d0 = jnp.sqrt(jnp.sum((obs14 - obs13) ** 2, axis=1, keepdims=True))
    d1 = jnp.sqrt(jnp.sum((obs13 - obs12) ** 2, axis=1, keepdims=True))
    d2 = jnp.sqrt(jnp.sum((obs12 - obs11) ** 2, axis=1, keepdims=True))
    rate = (d0 + d1 + d2) / 3.0

    act13 = a_t(13)
    act_pad = jnp.concatenate(
        [act13, jnp.zeros((BB, _OBS - _ACT), jnp.float32)], axis=1)
    oc = obs14 - jnp.mean(obs14, axis=1, keepdims=True)
    ac = act_pad - jnp.mean(act_pad, axis=1, keepdims=True)
    denom = (jnp.sqrt(jnp.sum(oc * oc, axis=1, keepdims=True)) *
             jnp.sqrt(jnp.sum(ac * ac, axis=1, keepdims=True)) + 1e-8)
    corr = jnp.sum(oc * ac, axis=1, keepdims=True) / denom

    xs = [jnp.concatenate([o_t(t), a_t(t)], axis=1) for t in range(_T)]
    X_all = jnp.concatenate(xs, axis=0)

    GI = _dot(X_all[0:15 * BB, :], dtw_WihT[...]) + dtw_bih[...]
    W = dtw_WhhT[...]
    bhh = dtw_bhh[...]
    h = jnp.zeros((BB, _GH), jnp.float32)
    for t in range(15):
        h = _gru_step(h, GI[t * BB:(t + 1) * BB, :], W, bhh, _GH)

    feats = jnp.concatenate([entropy, rate, corr, h], axis=1)
    mu_f = jnp.mean(feats, axis=1, keepdims=True)
    var_f = jnp.mean((feats - mu_f) ** 2, axis=1, keepdims=True)
    fn = (feats - mu_f) / jnp.sqrt(var_f + 1e-5) * ln_g[...] + ln_b[...]
    h1 = jnp.maximum(_dot(fn, m1_WT[...]) + m1_b[...], 0.0)
    h2 = jnp.maximum(_dot(h1, m2_WT[...]) + m2_b[...], 0.0)
    logits = _dot(h2, m3_WT[...]) + m3_b[...]
    lmx = jnp.max(logits, axis=1, keepdims=True)
    iota = jax.lax.broadcasted_iota(jnp.int32, (BB, 14), 1)
    widx = jnp.min(jnp.where(logits == lmx, iota, 14), axis=1, keepdims=True)
    s_off = (widx + 1) // 2
    e_off = (widx + 2) // 2

    GIf = _dot(X_all[7 * BB:15 * BB, :], f_WihT[...]) + f_bih[...]
    GIb = _dot(X_all[14 * BB:22 * BB, :], bk_WihT[...]) + bk_bih[...]
    fW = f_WhhT[...]
    fb = f_bhh[...]
    bW = bk_WhhT[...]
    bb = bk_bhh[...]
    hf = jnp.zeros((BB, _EMB), jnp.float32)
    hb = jnp.zeros((BB, _EMB), jnp.float32)
    for i in range(8):
        hf_new = _gru_step(hf, GIf[i * BB:(i + 1) * BB, :], fW, fb, _EMB)
        hf = jnp.where(s_off >= (7 - i), hf_new, hf)
        hb_new = _gru_step(hb, GIb[(7 - i) * BB:(8 - i) * BB, :], bW, bb, _EMB)
        hb = jnp.where(e_off >= (7 - i), hb_new, hb)

    combined = jnp.concatenate([hf, hb], axis=1)
    z1 = jnp.maximum(_dot(combined, e1_WT[...]) + e1_b[...], 0.0)
    z_enc = _dot(z1, e2_WT[...]) + e2_b[...]
    mu = _dot(z_enc, mu_WT[...]) + mu_b[...]
    lv = _dot(z_enc, lv_WT[...]) + lv_b[...]
    mu_ref[...] = mu
    sigma_ref[...] = jnp.exp(0.5 * lv)


def kernel(obs_chunk, act_chunk, dtw_Wih, dtw_Whh, dtw_bih, dtw_bhh,
           ln_g, ln_b, m1_W, m1_b, m2_W, m2_b, m3_W, m3_b,
           f_Wih, f_Whh, f_bih, f_bhh, bk_Wih, bk_Whh, bk_bih, bk_bhh,
           e1_W, e1_b, e2_W, e2_b, mu_W, mu_b, lv_W, lv_b, test_mode):
    B = obs_chunk.shape[0]
    obs2 = obs_chunk.reshape(B, _T * _OBS)
    act2 = act_chunk.reshape(B, _T * _ACT)

    weights = [
        dtw_Wih.T, dtw_Whh.T, dtw_bih.reshape(1, -1), dtw_bhh.reshape(1, -1),
        ln_g.reshape(1, -1), ln_b.reshape(1, -1),
        m1_W.T, m1_b.reshape(1, -1), m2_W.T, m2_b.reshape(1, -1),
        m3_W.T, m3_b.reshape(1, -1),
        f_Wih.T, f_Whh.T, f_bih.reshape(1, -1), f_bhh.reshape(1, -1),
        bk_Wih.T, bk_Whh.T, bk_bih.reshape(1, -1), bk_bhh.reshape(1, -1),
        e1_W.T, e1_b.reshape(1, -1), e2_W.T, e2_b.reshape(1, -1),
        mu_W.T, mu_b.reshape(1, -1), lv_W.T, lv_b.reshape(1, -1),
    ]

    grid = (B // _BB,)
    row_in = [
        pl.BlockSpec((_BB, _T * _OBS), lambda i: (i, 0)),
        pl.BlockSpec((_BB, _T * _ACT), lambda i: (i, 0)),
    ]
    w_specs = [pl.BlockSpec(w.shape, lambda i: (0, 0)) for w in weights]
    out_specs = [
        pl.BlockSpec((_BB, _EMB), lambda i: (i, 0)),
        pl.BlockSpec((_BB, _EMB), lambda i: (i, 0)),
    ]
    mu, sigma = pl.pallas_call(
        _block_kernel,
        grid=grid,
        in_specs=row_in + w_specs,
        out_specs=out_specs,
        out_shape=[
            jax.ShapeDtypeStruct((B, _EMB), jnp.float32),
            jax.ShapeDtypeStruct((B, _EMB), jnp.float32),
        ],
    )(obs2, act2, *weights)
    return (mu, mu, sigma)


# speedup vs baseline: 1.6825x; 1.0282x over previous
import jax
import jax.numpy as jnp
from jax.experimental import pallas as pl

_T = 22
_OBS = 64
_ACT = 16
_GH = 32
_EMB = 64
_BB = 1024

def _dot(a, b):
    return jax.lax.dot_general(a.astype(jnp.bfloat16), b.astype(jnp.bfloat16),
                               (((1,), (0,)), ((), ())),
                               preferred_element_type=jnp.float32)


def _gru_step(h, gi, W, bhh, G):
    gh = _dot(h, W) + bhh
    r = jax.nn.sigmoid(gi[:, :G] + gh[:, :G])
    z = jax.nn.sigmoid(gi[:, G:2 * G] + gh[:, G:2 * G])
    n = jnp.tanh(gi[:, 2 * G:] + r * gh[:, 2 * G:])
    return (1.0 - z) * n + z * h


def _block_kernel(obs_ref, act_ref,
                  dtw_WihT, dtw_WhhT, dtw_bih, dtw_bhh,
                  ln_g, ln_b, m1_WT, m1_b, m2_WT, m2_b, m3_WT, m3_b,
                  f_WihT, f_WhhT, f_bih, f_bhh,
                  bk_WihT, bk_WhhT, bk_bih, bk_bhh,
                  e1_WT, e1_b, e2_WT, e2_b, mu_WT, mu_b, lv_WT, lv_b,
                  mu_ref, sigma_ref):
    BB = _BB

    def o_t(t):
        return obs_ref[:, _OBS * t:_OBS * (t + 1)]

    def a_t(t):
        return act_ref[:, _ACT * t:_ACT * (t + 1)]

    obs14 = o_t(14)
    obs13 = o_t(13)
    obs12 = o_t(12)
    obs11 = o_t(11)

    mx = jnp.max(obs14, axis=1, keepdims=True)
    ex = jnp.exp(obs14 - mx)
    p = ex / jnp.sum(ex, axis=1, keepdims=True)
    entropy = -jnp.sum(p * jnp.log(p + 1e-8), axis=1, keepdims=True)

    d0 = jnp.sqrt(jnp.sum((obs14 - obs13) ** 2, axis=1, keepdims=True))
    d1 = jnp.sqrt(jnp.sum((obs13 - obs12) ** 2, axis=1, keepdims=True))
    d2 = jnp.sqrt(jnp.sum((obs12 - obs11) ** 2, axis=1, keepdims=True))
    rate = (d0 + d1 + d2) / 3.0

    act13 = a_t(13)
    act_pad = jnp.concatenate(
        [act13, jnp.zeros((BB, _OBS - _ACT), jnp.float32)], axis=1)
    oc = obs14 - jnp.mean(obs14, axis=1, keepdims=True)
    ac = act_pad - jnp.mean(act_pad, axis=1, keepdims=True)
    denom = (jnp.sqrt(jnp.sum(oc * oc, axis=1, keepdims=True)) *
             jnp.sqrt(jnp.sum(ac * ac, axis=1, keepdims=True)) + 1e-8)
    corr = jnp.sum(oc * ac, axis=1, keepdims=True) / denom

    xs = [jnp.concatenate([o_t(t), a_t(t)], axis=1) for t in range(_T)]
    X_all = jnp.concatenate(xs, axis=0).astype(jnp.bfloat16)

    GI = _dot(X_all[0:15 * BB, :], dtw_WihT[...]) + dtw_bih[...]
    W = dtw_WhhT[...]
    bhh = dtw_bhh[...]
    h = jnp.zeros((BB, _GH), jnp.float32)
    for t in range(15):
        h = _gru_step(h, GI[t * BB:(t + 1) * BB, :], W, bhh, _GH)

    feats = jnp.concatenate([entropy, rate, corr, h], axis=1)
    mu_f = jnp.mean(feats, axis=1, keepdims=True)
    var_f = jnp.mean((feats - mu_f) ** 2, axis=1, keepdims=True)
    fn = (feats - mu_f) / jnp.sqrt(var_f + 1e-5) * ln_g[...] + ln_b[...]
    h1 = jnp.maximum(_dot(fn, m1_WT[...]) + m1_b[...], 0.0)
    h2 = jnp.maximum(_dot(h1, m2_WT[...]) + m2_b[...], 0.0)
    logits = _dot(h2, m3_WT[...]) + m3_b[...]
    lmx = jnp.max(logits, axis=1, keepdims=True)
    iota = jax.lax.broadcasted_iota(jnp.int32, (BB, 14), 1)
    widx = jnp.min(jnp.where(logits == lmx, iota, 14), axis=1, keepdims=True)
    s_off = (widx + 1) // 2
    e_off = (widx + 2) // 2

    GIf = _dot(X_all[7 * BB:15 * BB, :], f_WihT[...]) + f_bih[...]
    GIb = _dot(X_all[14 * BB:22 * BB, :], bk_WihT[...]) + bk_bih[...]
    fW = f_WhhT[...]
    fb = f_bhh[...]
    bW = bk_WhhT[...]
    bb = bk_bhh[...]
    hf = jnp.zeros((BB, _EMB), jnp.float32)
    hb = jnp.zeros((BB, _EMB), jnp.float32)
    for i in range(8):
        hf_new = _gru_step(hf, GIf[i * BB:(i + 1) * BB, :], fW, fb, _EMB)
        hf = jnp.where(s_off >= (7 - i), hf_new, hf)
        hb_new = _gru_step(hb, GIb[(7 - i) * BB:(8 - i) * BB, :], bW, bb, _EMB)
        hb = jnp.where(e_off >= (7 - i), hb_new, hb)

    combined = jnp.concatenate([hf, hb], axis=1)
    z1 = jnp.maximum(_dot(combined, e1_WT[...]) + e1_b[...], 0.0)
    z_enc = _dot(z1, e2_WT[...]) + e2_b[...]
    mu = _dot(z_enc, mu_WT[...]) + mu_b[...]
    lv = _dot(z_enc, lv_WT[...]) + lv_b[...]
    mu_ref[...] = mu
    sigma_ref[...] = jnp.exp(0.5 * lv)


def kernel(obs_chunk, act_chunk, dtw_Wih, dtw_Whh, dtw_bih, dtw_bhh,
           ln_g, ln_b, m1_W, m1_b, m2_W, m2_b, m3_W, m3_b,
           f_Wih, f_Whh, f_bih, f_bhh, bk_Wih, bk_Whh, bk_bih, bk_bhh,
           e1_W, e1_b, e2_W, e2_b, mu_W, mu_b, lv_W, lv_b, test_mode):
    B = obs_chunk.shape[0]
    obs2 = obs_chunk.reshape(B, _T * _OBS)
    act2 = act_chunk.reshape(B, _T * _ACT)

    bf = jnp.bfloat16
    weights = [
        dtw_Wih.T.astype(bf), dtw_Whh.T.astype(bf),
        dtw_bih.reshape(1, -1), dtw_bhh.reshape(1, -1),
        ln_g.reshape(1, -1), ln_b.reshape(1, -1),
        m1_W.T.astype(bf), m1_b.reshape(1, -1), m2_W.T.astype(bf),
        m2_b.reshape(1, -1), m3_W.T.astype(bf), m3_b.reshape(1, -1),
        f_Wih.T.astype(bf), f_Whh.T.astype(bf),
        f_bih.reshape(1, -1), f_bhh.reshape(1, -1),
        bk_Wih.T.astype(bf), bk_Whh.T.astype(bf),
        bk_bih.reshape(1, -1), bk_bhh.reshape(1, -1),
        e1_W.T.astype(bf), e1_b.reshape(1, -1), e2_W.T.astype(bf),
        e2_b.reshape(1, -1), mu_W.T.astype(bf), mu_b.reshape(1, -1),
        lv_W.T.astype(bf), lv_b.reshape(1, -1),
    ]

    grid = (B // _BB,)
    row_in = [
        pl.BlockSpec((_BB, _T * _OBS), lambda i: (i, 0)),
        pl.BlockSpec((_BB, _T * _ACT), lambda i: (i, 0)),
    ]
    w_specs = [pl.BlockSpec(w.shape, lambda i: (0, 0)) for w in weights]
    out_specs = [
        pl.BlockSpec((_BB, _EMB), lambda i: (i, 0)),
        pl.BlockSpec((_BB, _EMB), lambda i: (i, 0)),
    ]
    mu, sigma = pl.pallas_call(
        _block_kernel,
        grid=grid,
        in_specs=row_in + w_specs,
        out_specs=out_specs,
        out_shape=[
            jax.ShapeDtypeStruct((B, _EMB), jnp.float32),
            jax.ShapeDtypeStruct((B, _EMB), jnp.float32),
        ],
    )(obs2, act2, *weights)
    return (mu, mu, sigma)
